# baseline (speedup 1.0000x reference)
import functools

import jax
import jax.numpy as jnp
from jax import lax
from jax.experimental import pallas as pl
from jax.experimental.pallas import tpu as pltpu

try:
    from jax.experimental.pallas import tpu_sc as plsc
    _HAS_SC = True
except ImportError:
    plsc = None
    _HAS_SC = False

EMB = 128
BATCH = 16384
NC = 2
NS = 16
NW = NC * NS
CHUNK = 128
NSLOT = 7


def _sc_gather(rid, pid, R_emb, P_emb, n, offset):
    bpw = n // NW
    nchunk = bpw // CHUNK
    nstep = 2 * nchunk
    nslot = min(NSLOT, nstep)
    mesh = plsc.VectorSubcoreMesh(core_axis_name="c", subcore_axis_name="s")

    @functools.partial(
        pl.kernel,
        mesh=mesh,
        out_type=[
            jax.ShapeDtypeStruct((n, EMB), jnp.float32),
            jax.ShapeDtypeStruct((n, EMB), jnp.float32),
        ],
        scratch_types=[
            pltpu.VMEM((bpw,), jnp.int32),
            pltpu.VMEM((bpw,), jnp.int32),
            pltpu.VMEM((nslot * CHUNK, EMB), jnp.float32),
            pltpu.SemaphoreType.DMA,
            pltpu.SemaphoreType.DMA,
        ],
    )
    def gather_k(rid_hbm, pid_hbm, R_hbm, P_hbm, rev_out, prod_out,
                 ridx_v, pidx_v, rows_v, gsem, osem):
        wid = lax.axis_index("s") * NC + lax.axis_index("c")
        base = wid * bpw

        i1 = pltpu.async_copy(rid_hbm.at[pl.ds(offset + base, bpw)],
                              ridx_v, gsem)
        i2 = pltpu.async_copy(pid_hbm.at[pl.ds(offset + base, bpw)],
                              pidx_v, gsem)
        i1.wait()
        i2.wait()

        def fire(k):
            slot = rows_v.at[pl.ds((k % nslot) * CHUNK, CHUNK)]
            if k < nchunk:
                idx = ridx_v.at[pl.ds(k * CHUNK, CHUNK)]
                return pltpu.async_copy(R_hbm.at[idx], slot, gsem)
            idx = pidx_v.at[pl.ds((k - nchunk) * CHUNK, CHUNK)]
            return pltpu.async_copy(P_hbm.at[idx], slot, gsem)

        def fire_out(k):
            slot = rows_v.at[pl.ds((k % nslot) * CHUNK, CHUNK)]
            if k < nchunk:
                dst = rev_out.at[pl.ds(base + k * CHUNK, CHUNK)]
            else:
                dst = prod_out.at[pl.ds(base + (k - nchunk) * CHUNK, CHUNK)]
            return pltpu.async_copy(slot, dst, osem)

        gathers = [fire(k) for k in range(nslot)]
        outs = []
        for k in range(nstep):
            if k >= nslot:
                outs[k - nslot].wait()
                gathers.append(fire(k))
            gathers[k].wait()
            outs.append(fire_out(k))
        for k in range(max(0, nstep - nslot), nstep):
            outs[k].wait()

    return gather_k(rid, pid, R_emb, P_emb)


def _mlp_body(rev_ref, prod_ref, w1r_ref, w1p_ref, b1_ref, w2_ref,
              b2_ref, out_ref):
    hT = lax.dot_general(w1r_ref[...], rev_ref[...],
                         (((0,), (1,)), ((), ())),
                         preferred_element_type=jnp.float32)
    hT = hT + lax.dot_general(w1p_ref[...], prod_ref[...],
                              (((0,), (1,)), ((), ())),
                              preferred_element_type=jnp.float32)
    hT = jnp.maximum(hT + b1_ref[...], 0.0)
    out_ref[...] = jnp.sum(hT * w2_ref[...], axis=0) + b2_ref[0, 0]


def _tc_mlp(rev, prod, w1r, w1p, b1c, w2c, b2r, n, block):
    grid = (n // block,)
    return pl.pallas_call(
        _mlp_body,
        grid=grid,
        in_specs=[
            pl.BlockSpec((block, EMB), lambda i: (i, 0)),
            pl.BlockSpec((block, EMB), lambda i: (i, 0)),
            pl.BlockSpec((EMB, 64), lambda i: (0, 0)),
            pl.BlockSpec((EMB, 64), lambda i: (0, 0)),
            pl.BlockSpec((64, 1), lambda i: (0, 0)),
            pl.BlockSpec((64, 1), lambda i: (0, 0)),
            pl.BlockSpec(memory_space=pltpu.SMEM),
        ],
        out_specs=pl.BlockSpec((block,), lambda i: (i,)),
        out_shape=jax.ShapeDtypeStruct((n,), jnp.float32),
    )(rev, prod, w1r, w1p, b1c, w2c, b2r)


SPLITS = (8192, 8192)
MLP_BLOCK = 2048


def kernel(product_id, reviewer_id, R_emb, P_emb, W1, b1, W2, b2):
    rid = reviewer_id.astype(jnp.int32)
    pid = product_id.astype(jnp.int32)
    w1r = W1[:EMB]
    w1p = W1[EMB:]
    b1c = b1.reshape(64, 1)
    b2r = b2.reshape(1, 1)
    gathered = []
    off = 0
    for n in SPLITS:
        gathered.append(_sc_gather(rid, pid, R_emb, P_emb, n, off))
        off += n
    outs = [
        _tc_mlp(rev, prod, w1r, w1p, b1c, W2, b2r, n, MLP_BLOCK)
        for n, (rev, prod) in zip(SPLITS, gathered)
    ]
    return jnp.concatenate(outs)


# speedup vs baseline: 1.0556x; 1.0556x over previous
import functools

import jax
import jax.numpy as jnp
from jax import lax
from jax.experimental import pallas as pl
from jax.experimental.pallas import tpu as pltpu

try:
    from jax.experimental.pallas import tpu_sc as plsc
    _HAS_SC = True
except ImportError:
    plsc = None
    _HAS_SC = False

EMB = 128
BATCH = 16384
NC = 2
NS = 16
NW = NC * NS
CHUNK = 128
NSLOT = 7


def _sc_gather(rid, pid, R_emb, P_emb, n, offset):
    bpw = n // NW
    nchunk = bpw // CHUNK
    nstep = 2 * nchunk
    nslot = min(NSLOT, nstep)
    mesh = plsc.VectorSubcoreMesh(core_axis_name="c", subcore_axis_name="s")

    @functools.partial(
        pl.kernel,
        mesh=mesh,
        out_type=[
            jax.ShapeDtypeStruct((n, EMB), jnp.float32),
            jax.ShapeDtypeStruct((n, EMB), jnp.float32),
        ],
        scratch_types=[
            pltpu.VMEM((bpw,), jnp.int32),
            pltpu.VMEM((bpw,), jnp.int32),
            pltpu.VMEM((nslot * CHUNK, EMB), jnp.float32),
            pltpu.SemaphoreType.DMA,
            pltpu.SemaphoreType.DMA,
        ],
    )
    def gather_k(rid_hbm, pid_hbm, R_hbm, P_hbm, rev_out, prod_out,
                 ridx_v, pidx_v, rows_v, gsem, osem):
        wid = lax.axis_index("s") * NC + lax.axis_index("c")
        base = wid * bpw

        i1 = pltpu.async_copy(rid_hbm.at[pl.ds(offset + base, bpw)],
                              ridx_v, gsem)
        i2 = pltpu.async_copy(pid_hbm.at[pl.ds(offset + base, bpw)],
                              pidx_v, gsem)
        i1.wait()
        i2.wait()

        def fire(k):
            slot = rows_v.at[pl.ds((k % nslot) * CHUNK, CHUNK)]
            if k < nchunk:
                idx = ridx_v.at[pl.ds(k * CHUNK, CHUNK)]
                return pltpu.async_copy(R_hbm.at[idx], slot, gsem)
            idx = pidx_v.at[pl.ds((k - nchunk) * CHUNK, CHUNK)]
            return pltpu.async_copy(P_hbm.at[idx], slot, gsem)

        def fire_out(k):
            slot = rows_v.at[pl.ds((k % nslot) * CHUNK, CHUNK)]
            if k < nchunk:
                dst = rev_out.at[pl.ds(base + k * CHUNK, CHUNK)]
            else:
                dst = prod_out.at[pl.ds(base + (k - nchunk) * CHUNK, CHUNK)]
            return pltpu.async_copy(slot, dst, osem)

        gathers = [fire(k) for k in range(nslot)]
        outs = []
        for k in range(nstep):
            if k >= nslot:
                outs[k - nslot].wait()
                gathers.append(fire(k))
            gathers[k].wait()
            outs.append(fire_out(k))
        for k in range(max(0, nstep - nslot), nstep):
            outs[k].wait()

    return gather_k(rid, pid, R_emb, P_emb)


def _mlp_body(rev_ref, prod_ref, w1r_ref, w1p_ref, b1_ref, w2_ref,
              b2_ref, out_ref):
    hT = lax.dot_general(w1r_ref[...], rev_ref[...],
                         (((0,), (1,)), ((), ())),
                         preferred_element_type=jnp.float32)
    hT = hT + lax.dot_general(w1p_ref[...], prod_ref[...],
                              (((0,), (1,)), ((), ())),
                              preferred_element_type=jnp.float32)
    hT = jnp.maximum(hT + b1_ref[...], 0.0)
    out_ref[...] = jnp.sum(hT * w2_ref[...], axis=0) + b2_ref[0, 0]


def _tc_mlp(rev, prod, w1r, w1p, b1c, w2c, b2r, n, block):
    grid = (n // block,)
    return pl.pallas_call(
        _mlp_body,
        grid=grid,
        in_specs=[
            pl.BlockSpec((block, EMB), lambda i: (i, 0)),
            pl.BlockSpec((block, EMB), lambda i: (i, 0)),
            pl.BlockSpec((EMB, 64), lambda i: (0, 0)),
            pl.BlockSpec((EMB, 64), lambda i: (0, 0)),
            pl.BlockSpec((64, 1), lambda i: (0, 0)),
            pl.BlockSpec((64, 1), lambda i: (0, 0)),
            pl.BlockSpec(memory_space=pltpu.SMEM),
        ],
        out_specs=pl.BlockSpec((block,), lambda i: (i,)),
        out_shape=jax.ShapeDtypeStruct((n,), jnp.float32),
    )(rev, prod, w1r, w1p, b1c, w2c, b2r)


SPLITS = (16384,)
MLP_BLOCK = 2048


def kernel(product_id, reviewer_id, R_emb, P_emb, W1, b1, W2, b2):
    rid = reviewer_id.astype(jnp.int32)
    pid = product_id.astype(jnp.int32)
    w1r = W1[:EMB]
    w1p = W1[EMB:]
    b1c = b1.reshape(64, 1)
    b2r = b2.reshape(1, 1)
    gathered = []
    off = 0
    for n in SPLITS:
        gathered.append(_sc_gather(rid, pid, R_emb, P_emb, n, off))
        off += n
    outs = [
        _tc_mlp(rev, prod, w1r, w1p, b1c, W2, b2r, n, MLP_BLOCK)
        for n, (rev, prod) in zip(SPLITS, gathered)
    ]
    return jnp.concatenate(outs)


# speedup vs baseline: 1.0634x; 1.0074x over previous
import functools

import jax
import jax.numpy as jnp
from jax import lax
from jax.experimental import pallas as pl
from jax.experimental.pallas import tpu as pltpu

try:
    from jax.experimental.pallas import tpu_sc as plsc
    _HAS_SC = True
except ImportError:
    plsc = None
    _HAS_SC = False

EMB = 128
BATCH = 16384
NC = 2
NS = 16
NW = NC * NS
CHUNK = 256
NSLOT = 3


def _sc_gather(rid, pid, R_emb, P_emb, n, offset):
    bpw = n // NW
    nchunk = bpw // CHUNK
    nstep = 2 * nchunk
    nslot = min(NSLOT, nstep)
    mesh = plsc.VectorSubcoreMesh(core_axis_name="c", subcore_axis_name="s")

    @functools.partial(
        pl.kernel,
        mesh=mesh,
        out_type=[
            jax.ShapeDtypeStruct((n, EMB), jnp.float32),
            jax.ShapeDtypeStruct((n, EMB), jnp.float32),
        ],
        scratch_types=[
            pltpu.VMEM((bpw,), jnp.int32),
            pltpu.VMEM((bpw,), jnp.int32),
            pltpu.VMEM((nslot * CHUNK, EMB), jnp.float32),
            pltpu.SemaphoreType.DMA,
            pltpu.SemaphoreType.DMA,
        ],
    )
    def gather_k(rid_hbm, pid_hbm, R_hbm, P_hbm, rev_out, prod_out,
                 ridx_v, pidx_v, rows_v, gsem, osem):
        wid = lax.axis_index("s") * NC + lax.axis_index("c")
        base = wid * bpw

        i1 = pltpu.async_copy(rid_hbm.at[pl.ds(offset + base, bpw)],
                              ridx_v, gsem)
        i2 = pltpu.async_copy(pid_hbm.at[pl.ds(offset + base, bpw)],
                              pidx_v, gsem)
        i1.wait()
        i2.wait()

        def fire(k):
            slot = rows_v.at[pl.ds((k % nslot) * CHUNK, CHUNK)]
            if k < nchunk:
                idx = ridx_v.at[pl.ds(k * CHUNK, CHUNK)]
                return pltpu.async_copy(R_hbm.at[idx], slot, gsem)
            idx = pidx_v.at[pl.ds((k - nchunk) * CHUNK, CHUNK)]
            return pltpu.async_copy(P_hbm.at[idx], slot, gsem)

        def fire_out(k):
            slot = rows_v.at[pl.ds((k % nslot) * CHUNK, CHUNK)]
            if k < nchunk:
                dst = rev_out.at[pl.ds(base + k * CHUNK, CHUNK)]
            else:
                dst = prod_out.at[pl.ds(base + (k - nchunk) * CHUNK, CHUNK)]
            return pltpu.async_copy(slot, dst, osem)

        gathers = [fire(k) for k in range(nslot)]
        outs = []
        for k in range(nstep):
            if k >= nslot:
                outs[k - nslot].wait()
                gathers.append(fire(k))
            gathers[k].wait()
            outs.append(fire_out(k))
        for k in range(max(0, nstep - nslot), nstep):
            outs[k].wait()

    return gather_k(rid, pid, R_emb, P_emb)


def _mlp_body(rev_ref, prod_ref, w1r_ref, w1p_ref, b1_ref, w2_ref,
              b2_ref, out_ref):
    hT = lax.dot_general(w1r_ref[...], rev_ref[...],
                         (((0,), (1,)), ((), ())),
                         preferred_element_type=jnp.float32)
    hT = hT + lax.dot_general(w1p_ref[...], prod_ref[...],
                              (((0,), (1,)), ((), ())),
                              preferred_element_type=jnp.float32)
    hT = jnp.maximum(hT + b1_ref[...], 0.0)
    out_ref[...] = jnp.sum(hT * w2_ref[...], axis=0) + b2_ref[0, 0]


def _tc_mlp(rev, prod, w1r, w1p, b1c, w2c, b2r, n, block):
    grid = (n // block,)
    return pl.pallas_call(
        _mlp_body,
        grid=grid,
        in_specs=[
            pl.BlockSpec((block, EMB), lambda i: (i, 0)),
            pl.BlockSpec((block, EMB), lambda i: (i, 0)),
            pl.BlockSpec((EMB, 64), lambda i: (0, 0)),
            pl.BlockSpec((EMB, 64), lambda i: (0, 0)),
            pl.BlockSpec((64, 1), lambda i: (0, 0)),
            pl.BlockSpec((64, 1), lambda i: (0, 0)),
            pl.BlockSpec(memory_space=pltpu.SMEM),
        ],
        out_specs=pl.BlockSpec((block,), lambda i: (i,)),
        out_shape=jax.ShapeDtypeStruct((n,), jnp.float32),
    )(rev, prod, w1r, w1p, b1c, w2c, b2r)


SPLITS = (16384,)
MLP_BLOCK = 2048


def kernel(product_id, reviewer_id, R_emb, P_emb, W1, b1, W2, b2):
    rid = reviewer_id.astype(jnp.int32)
    pid = product_id.astype(jnp.int32)
    w1r = W1[:EMB]
    w1p = W1[EMB:]
    b1c = b1.reshape(64, 1)
    b2r = b2.reshape(1, 1)
    gathered = []
    off = 0
    for n in SPLITS:
        gathered.append(_sc_gather(rid, pid, R_emb, P_emb, n, off))
        off += n
    outs = [
        _tc_mlp(rev, prod, w1r, w1p, b1c, W2, b2r, n, MLP_BLOCK)
        for n, (rev, prod) in zip(SPLITS, gathered)
    ]
    return jnp.concatenate(outs)


# speedup vs baseline: 1.0866x; 1.0218x over previous
import functools

import jax
import jax.numpy as jnp
from jax import lax
from jax.experimental import pallas as pl
from jax.experimental.pallas import tpu as pltpu

try:
    from jax.experimental.pallas import tpu_sc as plsc
    _HAS_SC = True
except ImportError:
    plsc = None
    _HAS_SC = False

EMB = 128
BATCH = 16384
NC = 2
NS = 16
NW = NC * NS
CHUNK = 256
NSLOT = 3


def _sc_gather(rid, pid, R_emb, P_emb, n, offset):
    bpw = n // NW
    nchunk = bpw // CHUNK
    nstep = 2 * nchunk
    nslot = min(NSLOT, nstep)
    mesh = plsc.VectorSubcoreMesh(core_axis_name="c", subcore_axis_name="s")

    @functools.partial(
        pl.kernel,
        mesh=mesh,
        out_type=[
            jax.ShapeDtypeStruct((n, EMB), jnp.float32),
            jax.ShapeDtypeStruct((n, EMB), jnp.float32),
        ],
        scratch_types=[
            pltpu.VMEM((bpw,), jnp.int32),
            pltpu.VMEM((bpw,), jnp.int32),
            pltpu.VMEM((nslot * CHUNK, EMB), jnp.float32),
            pltpu.SemaphoreType.DMA,
            pltpu.SemaphoreType.DMA,
        ],
    )
    def gather_k(rid_hbm, pid_hbm, R_hbm, P_hbm, rev_out, prod_out,
                 ridx_v, pidx_v, rows_v, gsem, osem):
        wid = lax.axis_index("s") * NC + lax.axis_index("c")
        base = wid * bpw

        i1 = pltpu.async_copy(rid_hbm.at[pl.ds(offset + base, bpw)],
                              ridx_v, gsem)
        i2 = pltpu.async_copy(pid_hbm.at[pl.ds(offset + base, bpw)],
                              pidx_v, gsem)
        i1.wait()
        pending_i2 = [i2]

        def fire(k):
            slot = rows_v.at[pl.ds((k % nslot) * CHUNK, CHUNK)]
            if k < nchunk:
                idx = ridx_v.at[pl.ds(k * CHUNK, CHUNK)]
                return pltpu.async_copy(R_hbm.at[idx], slot, gsem)
            if pending_i2:
                pending_i2.pop().wait()
            idx = pidx_v.at[pl.ds((k - nchunk) * CHUNK, CHUNK)]
            return pltpu.async_copy(P_hbm.at[idx], slot, gsem)

        def fire_out(k):
            slot = rows_v.at[pl.ds((k % nslot) * CHUNK, CHUNK)]
            if k < nchunk:
                dst = rev_out.at[pl.ds(base + k * CHUNK, CHUNK)]
            else:
                dst = prod_out.at[pl.ds(base + (k - nchunk) * CHUNK, CHUNK)]
            return pltpu.async_copy(slot, dst, osem)

        gathers = [fire(k) for k in range(nslot)]
        outs = []
        for k in range(nstep):
            if k >= nslot:
                outs[k - nslot].wait()
                gathers.append(fire(k))
            gathers[k].wait()
            outs.append(fire_out(k))
        for k in range(max(0, nstep - nslot), nstep):
            outs[k].wait()

    return gather_k(rid, pid, R_emb, P_emb)


def _mlp_body(rev_ref, prod_ref, w1r_ref, w1p_ref, b1_ref, w2_ref,
              b2_ref, out_ref):
    hT = lax.dot_general(w1r_ref[...], rev_ref[...],
                         (((0,), (1,)), ((), ())),
                         preferred_element_type=jnp.float32)
    hT = hT + lax.dot_general(w1p_ref[...], prod_ref[...],
                              (((0,), (1,)), ((), ())),
                              preferred_element_type=jnp.float32)
    hT = jnp.maximum(hT + b1_ref[...], 0.0)
    out_ref[...] = jnp.sum(hT * w2_ref[...], axis=0) + b2_ref[0, 0]


def _tc_mlp(rev, prod, w1r, w1p, b1c, w2c, b2r, n, block):
    grid = (n // block,)
    return pl.pallas_call(
        _mlp_body,
        grid=grid,
        in_specs=[
            pl.BlockSpec((block, EMB), lambda i: (i, 0)),
            pl.BlockSpec((block, EMB), lambda i: (i, 0)),
            pl.BlockSpec((EMB, 64), lambda i: (0, 0)),
            pl.BlockSpec((EMB, 64), lambda i: (0, 0)),
            pl.BlockSpec((64, 1), lambda i: (0, 0)),
            pl.BlockSpec((64, 1), lambda i: (0, 0)),
            pl.BlockSpec(memory_space=pltpu.SMEM),
        ],
        out_specs=pl.BlockSpec((block,), lambda i: (i,)),
        out_shape=jax.ShapeDtypeStruct((n,), jnp.float32),
    )(rev, prod, w1r, w1p, b1c, w2c, b2r)


SPLITS = (16384,)
MLP_BLOCK = 2048


def kernel(product_id, reviewer_id, R_emb, P_emb, W1, b1, W2, b2):
    rid = reviewer_id.astype(jnp.int32)
    pid = product_id.astype(jnp.int32)
    w1r = W1[:EMB]
    w1p = W1[EMB:]
    b1c = b1.reshape(64, 1)
    b2r = b2.reshape(1, 1)
    gathered = []
    off = 0
    for n in SPLITS:
        gathered.append(_sc_gather(rid, pid, R_emb, P_emb, n, off))
        off += n
    outs = [
        _tc_mlp(rev, prod, w1r, w1p, b1c, W2, b2r, n, MLP_BLOCK)
        for n, (rev, prod) in zip(SPLITS, gathered)
    ]
    return jnp.concatenate(outs)


# speedup vs baseline: 1.1167x; 1.0277x over previous
import functools

import jax
import jax.numpy as jnp
from jax import lax
from jax.experimental import pallas as pl
from jax.experimental.pallas import tpu as pltpu

try:
    from jax.experimental.pallas import tpu_sc as plsc
    _HAS_SC = True
except ImportError:
    plsc = None
    _HAS_SC = False

EMB = 128
BATCH = 16384
NC = 2
NS = 16
NW = NC * NS
CHUNK = 256
NSLOT = 3


def _sc_gather(rid, pid, R_emb, P_emb, n, offset):
    bpw = n // NW
    nchunk = bpw // CHUNK
    nstep = 2 * nchunk
    nslot = min(NSLOT, nstep)
    mesh = plsc.VectorSubcoreMesh(core_axis_name="c", subcore_axis_name="s")

    @functools.partial(
        pl.kernel,
        mesh=mesh,
        out_type=[
            jax.ShapeDtypeStruct((n, EMB), jnp.float32),
            jax.ShapeDtypeStruct((n, EMB), jnp.float32),
        ],
        scratch_types=[
            pltpu.VMEM((bpw,), jnp.int32),
            pltpu.VMEM((bpw,), jnp.int32),
            pltpu.VMEM((nslot * CHUNK, EMB), jnp.float32),
            pltpu.SemaphoreType.DMA,
            pltpu.SemaphoreType.DMA,
        ],
    )
    def gather_k(rid_hbm, pid_hbm, R_hbm, P_hbm, rev_out, prod_out,
                 ridx_v, pidx_v, rows_v, gsem, osem):
        wid = lax.axis_index("s") * NC + lax.axis_index("c")
        base = wid * bpw

        i1 = pltpu.async_copy(rid_hbm.at[pl.ds(offset + base, bpw)],
                              ridx_v, gsem)
        i2 = pltpu.async_copy(pid_hbm.at[pl.ds(offset + base, bpw)],
                              pidx_v, gsem)
        i1.wait()
        pending_i2 = [i2]

        def fire(k):
            slot = rows_v.at[pl.ds((k % nslot) * CHUNK, CHUNK)]
            if k < nchunk:
                idx = ridx_v.at[pl.ds(k * CHUNK, CHUNK)]
                return pltpu.async_copy(R_hbm.at[idx], slot, gsem)
            if pending_i2:
                pending_i2.pop().wait()
            idx = pidx_v.at[pl.ds((k - nchunk) * CHUNK, CHUNK)]
            return pltpu.async_copy(P_hbm.at[idx], slot, gsem)

        def fire_out(k):
            slot = rows_v.at[pl.ds((k % nslot) * CHUNK, CHUNK)]
            if k < nchunk:
                dst = rev_out.at[pl.ds(base + k * CHUNK, CHUNK)]
            else:
                dst = prod_out.at[pl.ds(base + (k - nchunk) * CHUNK, CHUNK)]
            return pltpu.async_copy(slot, dst, osem)

        gathers = [fire(k) for k in range(nslot)]
        outs = []
        for k in range(nstep):
            if k >= nslot:
                outs[k - nslot].wait()
                gathers.append(fire(k))
            gathers[k].wait()
            outs.append(fire_out(k))
        for k in range(max(0, nstep - nslot), nstep):
            outs[k].wait()

    return gather_k(rid, pid, R_emb, P_emb)


def _mlp_body(rev_ref, prod_ref, w1r_ref, w1p_ref, b1_ref, w2_ref,
              b2_ref, out_ref):
    hT = lax.dot_general(w1r_ref[...], rev_ref[...],
                         (((0,), (1,)), ((), ())),
                         preferred_element_type=jnp.float32)
    hT = hT + lax.dot_general(w1p_ref[...], prod_ref[...],
                              (((0,), (1,)), ((), ())),
                              preferred_element_type=jnp.float32)
    hT = jnp.maximum(hT + b1_ref[...], 0.0)
    out_ref[...] = jnp.sum(hT * w2_ref[...], axis=0) + b2_ref[0, 0]


def _tc_mlp(rev, prod, w1r, w1p, b1c, w2c, b2r, n, block):
    grid = (n // block,)
    return pl.pallas_call(
        _mlp_body,
        grid=grid,
        in_specs=[
            pl.BlockSpec((block, EMB), lambda i: (i, 0)),
            pl.BlockSpec((block, EMB), lambda i: (i, 0)),
            pl.BlockSpec((EMB, 64), lambda i: (0, 0)),
            pl.BlockSpec((EMB, 64), lambda i: (0, 0)),
            pl.BlockSpec((64, 1), lambda i: (0, 0)),
            pl.BlockSpec((64, 1), lambda i: (0, 0)),
            pl.BlockSpec(memory_space=pltpu.SMEM),
        ],
        out_specs=pl.BlockSpec((block,), lambda i: (i,)),
        out_shape=jax.ShapeDtypeStruct((n,), jnp.float32),
    )(rev, prod, w1r, w1p, b1c, w2c, b2r)


SPLITS = (16384,)
MLP_BLOCK = 4096


def kernel(product_id, reviewer_id, R_emb, P_emb, W1, b1, W2, b2):
    rid = reviewer_id.astype(jnp.int32)
    pid = product_id.astype(jnp.int32)
    w1r = W1[:EMB]
    w1p = W1[EMB:]
    b1c = b1.reshape(64, 1)
    b2r = b2.reshape(1, 1)
    gathered = []
    off = 0
    for n in SPLITS:
        gathered.append(_sc_gather(rid, pid, R_emb, P_emb, n, off))
        off += n
    outs = [
        _tc_mlp(rev, prod, w1r, w1p, b1c, W2, b2r, n, MLP_BLOCK)
        for n, (rev, prod) in zip(SPLITS, gathered)
    ]
    return jnp.concatenate(outs)


# speedup vs baseline: 1.1195x; 1.0025x over previous
import functools

import jax
import jax.numpy as jnp
from jax import lax
from jax.experimental import pallas as pl
from jax.experimental.pallas import tpu as pltpu

try:
    from jax.experimental.pallas import tpu_sc as plsc
    _HAS_SC = True
except ImportError:
    plsc = None
    _HAS_SC = False

EMB = 128
BATCH = 16384
NC = 2
NS = 16
NW = NC * NS
CHUNK = 256
NSLOT = 3


def _sc_gather(rid, pid, R_emb, P_emb, n, offset):
    bpw = n // NW
    nchunk = bpw // CHUNK
    nstep = 2 * nchunk
    nslot = min(NSLOT, nstep)
    mesh = plsc.VectorSubcoreMesh(core_axis_name="c", subcore_axis_name="s")

    @functools.partial(
        pl.kernel,
        mesh=mesh,
        out_type=[
            jax.ShapeDtypeStruct((n, EMB), jnp.float32),
            jax.ShapeDtypeStruct((n, EMB), jnp.float32),
        ],
        scratch_types=[
            pltpu.VMEM((bpw,), jnp.int32),
            pltpu.VMEM((bpw,), jnp.int32),
            pltpu.VMEM((nslot * CHUNK, EMB), jnp.float32),
            pltpu.SemaphoreType.DMA,
            pltpu.SemaphoreType.DMA,
        ],
    )
    def gather_k(rid_hbm, pid_hbm, R_hbm, P_hbm, rev_out, prod_out,
                 ridx_v, pidx_v, rows_v, gsem, osem):
        wid = lax.axis_index("s") * NC + lax.axis_index("c")
        base = wid * bpw

        i1 = pltpu.async_copy(rid_hbm.at[pl.ds(offset + base, bpw)],
                              ridx_v, gsem)
        i2 = pltpu.async_copy(pid_hbm.at[pl.ds(offset + base, bpw)],
                              pidx_v, gsem)
        i1.wait()
        pending_i2 = [i2]

        def fire(k):
            slot = rows_v.at[pl.ds((k % nslot) * CHUNK, CHUNK)]
            if k < nchunk:
                idx = ridx_v.at[pl.ds(k * CHUNK, CHUNK)]
                return pltpu.async_copy(R_hbm.at[idx], slot, gsem)
            if pending_i2:
                pending_i2.pop().wait()
            idx = pidx_v.at[pl.ds((k - nchunk) * CHUNK, CHUNK)]
            return pltpu.async_copy(P_hbm.at[idx], slot, gsem)

        def fire_out(k):
            slot = rows_v.at[pl.ds((k % nslot) * CHUNK, CHUNK)]
            if k < nchunk:
                dst = rev_out.at[pl.ds(base + k * CHUNK, CHUNK)]
            else:
                dst = prod_out.at[pl.ds(base + (k - nchunk) * CHUNK, CHUNK)]
            return pltpu.async_copy(slot, dst, osem)

        gathers = [fire(k) for k in range(nslot)]
        outs = []
        for k in range(nstep):
            if k >= nslot:
                outs[k - nslot].wait()
                gathers.append(fire(k))
            gathers[k].wait()
            outs.append(fire_out(k))
        for k in range(max(0, nstep - nslot), nstep):
            outs[k].wait()

    return gather_k(rid, pid, R_emb, P_emb)


def _mlp_body(rev_ref, prod_ref, w1r_ref, w1p_ref, b1_ref, w2_ref,
              b2_ref, out_ref):
    hT = lax.dot_general(w1r_ref[...], rev_ref[...],
                         (((0,), (1,)), ((), ())),
                         preferred_element_type=jnp.float32)
    hT = hT + lax.dot_general(w1p_ref[...], prod_ref[...],
                              (((0,), (1,)), ((), ())),
                              preferred_element_type=jnp.float32)
    hT = jnp.maximum(hT + b1_ref[...], 0.0)
    out_ref[...] = jnp.sum(hT * w2_ref[...], axis=0) + b2_ref[0, 0]


def _tc_mlp(rev, prod, w1r, w1p, b1c, w2c, b2r, n, block):
    grid = (n // block,)
    return pl.pallas_call(
        _mlp_body,
        grid=grid,
        in_specs=[
            pl.BlockSpec((block, EMB), lambda i: (i, 0)),
            pl.BlockSpec((block, EMB), lambda i: (i, 0)),
            pl.BlockSpec((EMB, 64), lambda i: (0, 0)),
            pl.BlockSpec((EMB, 64), lambda i: (0, 0)),
            pl.BlockSpec((64, 1), lambda i: (0, 0)),
            pl.BlockSpec((64, 1), lambda i: (0, 0)),
            pl.BlockSpec(memory_space=pltpu.SMEM),
        ],
        out_specs=pl.BlockSpec((block,), lambda i: (i,)),
        out_shape=jax.ShapeDtypeStruct((n,), jnp.float32),
    )(rev, prod, w1r, w1p, b1c, w2c, b2r)


SPLITS = (16384,)
MLP_BLOCK = 8192


def kernel(product_id, reviewer_id, R_emb, P_emb, W1, b1, W2, b2):
    rid = reviewer_id.astype(jnp.int32)
    pid = product_id.astype(jnp.int32)
    w1r = W1[:EMB]
    w1p = W1[EMB:]
    b1c = b1.reshape(64, 1)
    b2r = b2.reshape(1, 1)
    gathered = []
    off = 0
    for n in SPLITS:
        gathered.append(_sc_gather(rid, pid, R_emb, P_emb, n, off))
        off += n
    outs = [
        _tc_mlp(rev, prod, w1r, w1p, b1c, W2, b2r, n, MLP_BLOCK)
        for n, (rev, prod) in zip(SPLITS, gathered)
    ]
    return jnp.concatenate(outs)
